# baseline (speedup 1.0000x reference)
import functools

import jax
import jax.numpy as jnp
from jax import lax
from jax.experimental import pallas as pl
from jax.experimental.pallas import tpu as pltpu
from jax.experimental.pallas import tpu_sc as plsc

N_NODES = 10000
N_EDGES = 320000
D_FEAT = 128
N_CLASSES = 64

NC, NS = 2, 16
CB = N_CLASSES // NC
CHUNK = 128
NCH = 160
E_PK = NS * NCH * CHUNK
PADE = E_PK - N_EDGES
N_PAD = 10240
RPT = N_PAD // NS
DEPTH = 4
NG = NCH // DEPTH


def _mm_body(feat_ref, w_ref, o_ref):
    for h in range(NC):
        o_ref[h, :N_NODES] = lax.dot_general(
            feat_ref[...], w_ref[h * CB:(h + 1) * CB],
            (((1,), (1,)), ((), ())),
            preferred_element_type=jnp.float32,
        )
        o_ref[h, N_NODES:] = jnp.zeros((N_PAD - N_NODES, CB), jnp.float32)


def _tc_matmul(feat, W):
    return pl.pallas_call(
        _mm_body,
        out_shape=jax.ShapeDtypeStruct((NC, N_PAD, CB), jnp.float32),
    )(feat, W)


def _make_sc_sgc():
    mesh = plsc.VectorSubcoreMesh(core_axis_name="c", subcore_axis_name="s")
    scratch = (
        [pltpu.VMEM_SHARED((N_PAD, CB), jnp.float32)] * 2
        + [pltpu.VMEM((2, CHUNK), jnp.int32)] * DEPTH
        + [pltpu.VMEM((CHUNK, CB), jnp.float32)] * DEPTH
        + [pltpu.SemaphoreType.DMA] * (4 + 3 * DEPTH)
    )

    @functools.partial(
        pl.kernel,
        out_type=jax.ShapeDtypeStruct((NC, N_PAD, CB), jnp.float32),
        mesh=mesh,
        scratch_types=scratch,
        compiler_params=pltpu.CompilerParams(use_tc_tiling_on_sc=False),
    )
    def run(y_hbm, z_hbm, b_hbm, epk_hbm, out_hbm, y_sh, acc_sh, *rest):
        idx2 = rest[:DEPTH]
        rows = rest[DEPTH:2 * DEPTH]
        sem_y, sem_z, sem_b, sem_w = rest[2 * DEPTH:2 * DEPTH + 4]
        k = 2 * DEPTH + 4
        sem_i = rest[k:k + DEPTH]
        sem_g = rest[k + DEPTH:k + 2 * DEPTH]
        sem_s = rest[k + 2 * DEPTH:]

        cid = lax.axis_index("c")
        sid = lax.axis_index("s")
        r0 = sid * RPT

        def prefetch_first():
            for j in range(DEPTH):
                pltpu.async_copy(epk_hbm.at[sid, j], idx2[j], sem_i[j])

        def edge_pass(src_sh, dst_sh):
            def grp(g, carry):
                gd = []
                for j in range(DEPTH):
                    pltpu.make_async_copy(epk_hbm.at[sid, 0], idx2[j],
                                          sem_i[j]).wait()
                    gd.append(pltpu.async_copy(src_sh.at[idx2[j].at[0]],
                                               rows[j], sem_g[j]))
                sd = []
                for j in range(DEPTH):
                    gd[j].wait()
                    sd.append(pltpu.async_copy(rows[j],
                                               dst_sh.at[idx2[j].at[1]],
                                               sem_s[j], add=True))
                for j in range(DEPTH):
                    sd[j].wait()

                @pl.when(g < NG - 1)
                def _():
                    for j in range(DEPTH):
                        pltpu.async_copy(epk_hbm.at[sid, (g + 1) * DEPTH + j],
                                         idx2[j], sem_i[j])

                return carry

            lax.fori_loop(0, NG, grp, 0)

        dy = pltpu.async_copy(y_hbm.at[cid, pl.ds(r0, RPT)],
                              y_sh.at[pl.ds(r0, RPT)], sem_y)
        dz = pltpu.async_copy(z_hbm.at[pl.ds(r0, RPT)],
                              acc_sh.at[pl.ds(r0, RPT)], sem_z)
        prefetch_first()
        dy.wait()
        dz.wait()
        plsc.subcore_barrier()

        edge_pass(y_sh, acc_sh)
        plsc.subcore_barrier()

        db = pltpu.async_copy(b_hbm.at[cid, pl.ds(r0, RPT)],
                              y_sh.at[pl.ds(r0, RPT)], sem_b)
        prefetch_first()
        db.wait()
        plsc.subcore_barrier()

        edge_pass(acc_sh, y_sh)
        plsc.subcore_barrier()

        pltpu.async_copy(y_sh.at[pl.ds(r0, RPT)],
                         out_hbm.at[cid, pl.ds(r0, RPT)], sem_w).wait()

    return run


_sc_sgc = _make_sc_sgc()


def kernel(feat, edge_index, W, b):
    fill = jnp.arange(PADE, dtype=jnp.int32) % (N_PAD - N_NODES)
    srcp = jnp.concatenate([edge_index[0], fill])
    dstp = jnp.concatenate([edge_index[1], N_NODES + fill])
    epk = jnp.stack([srcp.reshape(NS, NCH, CHUNK),
                     dstp.reshape(NS, NCH, CHUNK)], axis=2)
    z = jnp.zeros((N_PAD, CB), jnp.float32)
    bimg = jnp.broadcast_to(b.reshape(NC, 1, CB), (NC, N_PAD, CB))

    y0 = _tc_matmul(feat, W)
    q = _sc_sgc(y0, z, bimg, epk)
    return jnp.concatenate([q[0, :N_NODES], q[1, :N_NODES]], axis=1)


# speedup vs baseline: 1.1088x; 1.1088x over previous
import functools

import jax
import jax.numpy as jnp
from jax import lax
from jax.experimental import pallas as pl
from jax.experimental.pallas import tpu as pltpu
from jax.experimental.pallas import tpu_sc as plsc

N_NODES = 10000
N_EDGES = 320000
D_FEAT = 128
N_CLASSES = 64

NC, NS = 2, 16
CB = N_CLASSES // NC
CHUNK = 128
NCH = 160
E_PK = NS * NCH * CHUNK
PADE = E_PK - N_EDGES
N_PAD = 10240
RPT = N_PAD // NS
DEPTH = 8
NG = NCH // DEPTH


def _mm_body(feat_ref, w_ref, o_ref):
    for h in range(NC):
        o_ref[h, :N_NODES] = lax.dot_general(
            feat_ref[...], w_ref[h * CB:(h + 1) * CB],
            (((1,), (1,)), ((), ())),
            preferred_element_type=jnp.float32,
        )
        o_ref[h, N_NODES:] = jnp.zeros((N_PAD - N_NODES, CB), jnp.float32)


def _tc_matmul(feat, W):
    return pl.pallas_call(
        _mm_body,
        out_shape=jax.ShapeDtypeStruct((NC, N_PAD, CB), jnp.float32),
    )(feat, W)


def _make_sc_sgc():
    mesh = plsc.VectorSubcoreMesh(core_axis_name="c", subcore_axis_name="s")
    scratch = (
        [pltpu.VMEM_SHARED((N_PAD, CB), jnp.float32)] * 2
        + [pltpu.VMEM((2, CHUNK), jnp.int32)] * DEPTH
        + [pltpu.VMEM((CHUNK, CB), jnp.float32)] * DEPTH
        + [pltpu.SemaphoreType.DMA] * (4 + 3 * DEPTH)
    )

    @functools.partial(
        pl.kernel,
        out_type=jax.ShapeDtypeStruct((NC, N_PAD, CB), jnp.float32),
        mesh=mesh,
        scratch_types=scratch,
        compiler_params=pltpu.CompilerParams(use_tc_tiling_on_sc=False),
    )
    def run(y_hbm, z_hbm, b_hbm, epk_hbm, out_hbm, y_sh, acc_sh, *rest):
        idx2 = rest[:DEPTH]
        rows = rest[DEPTH:2 * DEPTH]
        sem_y, sem_z, sem_b, sem_w = rest[2 * DEPTH:2 * DEPTH + 4]
        k = 2 * DEPTH + 4
        sem_i = rest[k:k + DEPTH]
        sem_g = rest[k + DEPTH:k + 2 * DEPTH]
        sem_s = rest[k + 2 * DEPTH:]

        cid = lax.axis_index("c")
        sid = lax.axis_index("s")
        r0 = sid * RPT

        def prefetch_first():
            for j in range(DEPTH):
                pltpu.async_copy(epk_hbm.at[sid, j], idx2[j], sem_i[j])

        def edge_pass(src_sh, dst_sh):
            def grp(g, carry):
                gd = []
                for j in range(DEPTH):
                    pltpu.make_async_copy(epk_hbm.at[sid, 0], idx2[j],
                                          sem_i[j]).wait()
                    gd.append(pltpu.async_copy(src_sh.at[idx2[j].at[0]],
                                               rows[j], sem_g[j]))
                sd = []
                for j in range(DEPTH):
                    gd[j].wait()
                    sd.append(pltpu.async_copy(rows[j],
                                               dst_sh.at[idx2[j].at[1]],
                                               sem_s[j], add=True))
                for j in range(DEPTH):
                    sd[j].wait()

                @pl.when(g < NG - 1)
                def _():
                    for j in range(DEPTH):
                        pltpu.async_copy(epk_hbm.at[sid, (g + 1) * DEPTH + j],
                                         idx2[j], sem_i[j])

                return carry

            lax.fori_loop(0, NG, grp, 0)

        dy = pltpu.async_copy(y_hbm.at[cid, pl.ds(r0, RPT)],
                              y_sh.at[pl.ds(r0, RPT)], sem_y)
        dz = pltpu.async_copy(z_hbm.at[pl.ds(r0, RPT)],
                              acc_sh.at[pl.ds(r0, RPT)], sem_z)
        prefetch_first()
        dy.wait()
        dz.wait()
        plsc.subcore_barrier()

        edge_pass(y_sh, acc_sh)
        plsc.subcore_barrier()

        db = pltpu.async_copy(b_hbm.at[cid, pl.ds(r0, RPT)],
                              y_sh.at[pl.ds(r0, RPT)], sem_b)
        prefetch_first()
        db.wait()
        plsc.subcore_barrier()

        edge_pass(acc_sh, y_sh)
        plsc.subcore_barrier()

        pltpu.async_copy(y_sh.at[pl.ds(r0, RPT)],
                         out_hbm.at[cid, pl.ds(r0, RPT)], sem_w).wait()

    return run


_sc_sgc = _make_sc_sgc()


def kernel(feat, edge_index, W, b):
    fill = jnp.arange(PADE, dtype=jnp.int32) % (N_PAD - N_NODES)
    srcp = jnp.concatenate([edge_index[0], fill])
    dstp = jnp.concatenate([edge_index[1], N_NODES + fill])
    epk = jnp.stack([srcp.reshape(NS, NCH, CHUNK),
                     dstp.reshape(NS, NCH, CHUNK)], axis=2)
    z = jnp.zeros((N_PAD, CB), jnp.float32)
    bimg = jnp.broadcast_to(b.reshape(NC, 1, CB), (NC, N_PAD, CB))

    y0 = _tc_matmul(feat, W)
    q = _sc_sgc(y0, z, bimg, epk)
    return jnp.concatenate([q[0, :N_NODES], q[1, :N_NODES]], axis=1)
